# baseline (speedup 1.0000x reference)
import functools

import jax
import jax.numpy as jnp
from jax import lax
from jax.experimental import pallas as pl
from jax.experimental.pallas import tpu as pltpu
from jax.experimental.pallas import tpu_sc as plsc

N_CLUSTERS = 100000
HIDDEN = 64
SEQ = 200

NUM_CORES = 2
NUM_SUBCORES = 16
NW = NUM_CORES * NUM_SUBCORES

CHUNK = 2 * SEQ
GSLICE = 80
NGS = CHUNK // GSLICE
TOTAL = 4096 * SEQ
NCHUNKS = TOTAL // CHUNK
CH_PER_W = NCHUNKS // NW
NBUF = 4
LOOK = 2


def _body(x_hbm, pix_hbm, pos_hbm, out_hbm,
          idx0, idx1, idx2, idx3, rows0, rows1, rows2, rows3, pos_v,
          sem_i0, sem_i1, sem_i2, sem_i3,
          sem_g0, sem_g1, sem_g2, sem_g3,
          sem_o0, sem_o1, sem_o2, sem_o3):
    wid = lax.axis_index("s") * NUM_CORES + lax.axis_index("c")
    base2 = wid * (2 * CH_PER_W)
    idx = (idx0, idx1, idx2, idx3)
    rows = (rows0, rows1, rows2, rows3)
    sem_i = (sem_i0, sem_i1, sem_i2, sem_i3)
    sem_g = (sem_g0, sem_g1, sem_g2, sem_g3)
    sem_o = (sem_o0, sem_o1, sem_o2, sem_o3)

    pltpu.sync_copy(pos_hbm.at[pl.ds(0, SEQ)], pos_v)

    def fire_idx(c, b):
        pltpu.async_copy(x_hbm.at[wid, c], idx[b], sem_i[b])

    def wait_idx(b):
        pltpu.make_async_copy(x_hbm.at[wid, 0], idx[b], sem_i[b]).wait()

    def fire_gathers(c, b):
        del c
        for k in range(NGS):
            pltpu.async_copy(
                pix_hbm.at[idx[b].at[pl.ds(k * GSLICE, GSLICE)]],
                rows[b].at[pl.ds(k * GSLICE, GSLICE)],
                sem_g[b],
            )

    def fire_out(c, b):
        pltpu.async_copy(rows[b].at[pl.ds(0, SEQ)], out_hbm.at[base2 + 2 * c], sem_o[b])
        pltpu.async_copy(rows[b].at[pl.ds(SEQ, SEQ)], out_hbm.at[base2 + 2 * c + 1], sem_o[b])

    def drain_out(c, b):
        pltpu.make_async_copy(
            rows[b].at[pl.ds(0, SEQ)], out_hbm.at[base2 + 2 * c], sem_o[b]
        ).wait()
        pltpu.make_async_copy(
            rows[b].at[pl.ds(SEQ, SEQ)], out_hbm.at[base2 + 2 * c + 1], sem_o[b]
        ).wait()

    def _chunk_step(c, b):
        bn = (b + LOOK) % NBUF
        bi = (b + LOOK + 1) % NBUF

        @pl.when(c + LOOK + 1 < CH_PER_W)
        def _fire_idx_ahead():
            fire_idx(c + LOOK + 1, bi)

        @pl.when(c + LOOK < CH_PER_W)
        def _fire_ahead():
            @pl.when(c + LOOK >= NBUF)
            def _drain_prev_out():
                drain_out(c + LOOK - NBUF, bn)
            wait_idx(bn)
            fire_gathers(c + LOOK, bn)

        pltpu.make_async_copy(
            pix_hbm.at[pl.ds(0, CHUNK)], rows[b], sem_g[b]
        ).wait()

        @pl.loop(0, SEQ, unroll=8)
        def _add(r):
            for j in range(HIDDEN // 16):
                sl = pl.ds(j * 16, 16)
                p = pos_v[r, sl]
                plsc.addupdate(rows[b].at[r, sl], p)
                plsc.addupdate(rows[b].at[SEQ + r, sl], p)

        fire_out(c, b)

    for p in range(LOOK + 1):
        fire_idx(p, p)
    for p in range(LOOK):
        wait_idx(p)
        fire_gathers(p, p)

    @pl.loop(0, CH_PER_W, step=NBUF)
    def _chunk_grp(c0):
        for b in range(NBUF):
            _chunk_step(c0 + b, b)

    for q in range(NBUF):
        c = CH_PER_W - NBUF + q
        drain_out(c, c % NBUF)


def kernel(x, pix_table, pos_table):
    b, seq = x.shape
    x3 = x.astype(jnp.int32).reshape(NW, CH_PER_W, CHUNK)
    mesh = plsc.VectorSubcoreMesh(core_axis_name="c", subcore_axis_name="s")
    run = functools.partial(
        pl.kernel,
        mesh=mesh,
        out_type=jax.ShapeDtypeStruct((4096, SEQ, HIDDEN), jnp.float32),
        scratch_types=[
            pltpu.VMEM((CHUNK,), jnp.int32),
            pltpu.VMEM((CHUNK,), jnp.int32),
            pltpu.VMEM((CHUNK,), jnp.int32),
            pltpu.VMEM((CHUNK,), jnp.int32),
            pltpu.VMEM((CHUNK, HIDDEN), jnp.float32),
            pltpu.VMEM((CHUNK, HIDDEN), jnp.float32),
            pltpu.VMEM((CHUNK, HIDDEN), jnp.float32),
            pltpu.VMEM((CHUNK, HIDDEN), jnp.float32),
            pltpu.VMEM((SEQ, HIDDEN), jnp.float32),
            pltpu.SemaphoreType.DMA,
            pltpu.SemaphoreType.DMA,
            pltpu.SemaphoreType.DMA,
            pltpu.SemaphoreType.DMA,
            pltpu.SemaphoreType.DMA,
            pltpu.SemaphoreType.DMA,
            pltpu.SemaphoreType.DMA,
            pltpu.SemaphoreType.DMA,
            pltpu.SemaphoreType.DMA,
            pltpu.SemaphoreType.DMA,
            pltpu.SemaphoreType.DMA,
            pltpu.SemaphoreType.DMA,
        ],
        compiler_params=pltpu.CompilerParams(use_tc_tiling_on_sc=False),
    )(_body)
    return run(x3, pix_table, pos_table)
